# baseline (speedup 1.0000x reference)
import functools

import jax
import jax.numpy as jnp
from jax import lax
from jax.experimental import pallas as pl
from jax.experimental.pallas import tpu as pltpu
from jax.experimental.pallas import tpu_sc as plsc

N = 10000
E = 160000
D = 256
ROW_BLK = 400

NC = 2
NS = 16
L = 16
NW = NC * NS
E_PAD = 163840
EPW = E_PAD // NW
G = 128
NCHUNK = EPW // G


DW = D // 2


def _s1_body(a_hbm, b_hbm, src_hbm, dst_hbm, w2e_hbm, w2o_hbm, b2_hbm,
             temp_hbm,
             masks_hbm, cnt_hbm,
             idx_s0, idx_d0, arow0, brow0, idx_s1, idx_d1, arow1, brow1,
             w2e_v, w2o_v, b2_v, temp_v, mask_v, cnt_v,
             sem_a0, sem_b0, sem_a1, sem_b1):
    wid = lax.axis_index("c") * NS + lax.axis_index("s")
    base = wid * EPW
    pltpu.sync_copy(w2e_hbm, w2e_v)
    pltpu.sync_copy(w2o_hbm, w2o_v)
    pltpu.sync_copy(b2_hbm, b2_v)
    pltpu.sync_copy(temp_hbm, temp_v)
    b2 = b2_v[...]
    et = jnp.exp(temp_v[...])
    ngrp = G // L
    eids = [lax.iota(jnp.int32, L) + (g * L) for g in range(ngrp)]

    def issue(c, idx_s, idx_d, arow, brow, sem_a, sem_b):
        off = base + c * G
        pltpu.sync_copy(src_hbm.at[pl.ds(off, G)], idx_s)
        pltpu.sync_copy(dst_hbm.at[pl.ds(off, G)], idx_d)
        pltpu.async_copy(a_hbm.at[idx_s], arow, sem_a)
        pltpu.async_copy(b_hbm.at[idx_d], brow, sem_b)

    def wait(idx_s, idx_d, arow, brow, sem_a, sem_b):
        pltpu.make_async_copy(a_hbm.at[idx_s], arow, sem_a).wait()
        pltpu.make_async_copy(b_hbm.at[idx_d], brow, sem_b).wait()

    def compute(c, arow, brow, cnt):
        off = base + c * G

        init = (tuple(jnp.zeros((L,), jnp.float32) for _ in range(ngrp)),
                jnp.zeros((L,), jnp.int32))

        @plsc.parallel_loop(0, DW, unroll=2, carry=init)
        def accs(k, carry):
            acc, kv = carry
            w2e = plsc.load_gather(w2e_v, [kv >> 4, kv & 15])
            w2o = plsc.load_gather(w2o_v, [kv >> 4, kv & 15])
            nacc = []
            for g in range(ngrp):
                a0, a1 = plsc.unpack(
                    plsc.bitcast(plsc.load_gather(arow, [eids[g], kv]),
                                 jnp.bfloat16),
                    format=plsc.PackFormat.INTERLEAVED)
                b0, b1 = plsc.unpack(
                    plsc.bitcast(plsc.load_gather(brow, [eids[g], kv]),
                                 jnp.bfloat16),
                    format=plsc.PackFormat.INTERLEAVED)
                nacc.append(acc[g]
                            + jnp.maximum(a0 + b0, 0.0) * w2e
                            + jnp.maximum(a1 + b1, 0.0) * w2o)
            return (tuple(nacc), kv + 1)

        accs = accs[0]

        for g in range(ngrp):
            logit = accs[g] + b2
            imp = 1.0 / (1.0 + jnp.exp(-logit))
            m = 1.0 / (1.0 + jnp.exp(-(imp - 0.5) * et))
            gid = off + g * L + lax.iota(jnp.int32, L)
            valid = gid < E
            m = jnp.where(valid, m, 0.0)
            mask_v[pl.ds(g * L, L)] = m
            cnt = cnt + jnp.where(valid & (m < 0.5), 1.0, 0.0)
        pltpu.sync_copy(mask_v, masks_hbm.at[pl.ds(off, G)])
        return cnt

    issue(0, idx_s0, idx_d0, arow0, brow0, sem_a0, sem_b0)

    def pair(i, cnt):
        c0 = 2 * i
        issue(c0 + 1, idx_s1, idx_d1, arow1, brow1, sem_a1, sem_b1)
        wait(idx_s0, idx_d0, arow0, brow0, sem_a0, sem_b0)
        cnt = compute(c0, arow0, brow0, cnt)

        @pl.when(c0 + 2 < NCHUNK)
        def _():
            issue(c0 + 2, idx_s0, idx_d0, arow0, brow0, sem_a0, sem_b0)

        wait(idx_s1, idx_d1, arow1, brow1, sem_a1, sem_b1)
        return compute(c0 + 1, arow1, brow1, cnt)

    cnt = lax.fori_loop(0, NCHUNK // 2, pair,
                        jnp.zeros((L,), jnp.float32))
    cnt_v[...] = cnt
    pltpu.sync_copy(cnt_v, cnt_hbm.at[wid])


def _s1(Au, Bu, srcp, dstp, w2e, w2o, b2v, tempv):
    mesh = plsc.VectorSubcoreMesh(core_axis_name="c", subcore_axis_name="s",
                                  num_cores=NC, num_subcores=NS)
    return pl.kernel(
        _s1_body,
        out_type=[jax.ShapeDtypeStruct((E_PAD,), jnp.float32),
                  jax.ShapeDtypeStruct((NW, L), jnp.float32)],
        mesh=mesh,
        compiler_params=pltpu.CompilerParams(needs_layout_passes=False,
                                             use_tc_tiling_on_sc=False),
        scratch_types=[
            pltpu.VMEM((G,), jnp.int32),
            pltpu.VMEM((G,), jnp.int32),
            pltpu.VMEM((G, DW), jnp.int32),
            pltpu.VMEM((G, DW), jnp.int32),
            pltpu.VMEM((G,), jnp.int32),
            pltpu.VMEM((G,), jnp.int32),
            pltpu.VMEM((G, DW), jnp.int32),
            pltpu.VMEM((G, DW), jnp.int32),
            pltpu.VMEM((DW // L, L), jnp.float32),
            pltpu.VMEM((DW // L, L), jnp.float32),
            pltpu.VMEM((L,), jnp.float32),
            pltpu.VMEM((L,), jnp.float32),
            pltpu.VMEM((G,), jnp.float32),
            pltpu.VMEM((L,), jnp.float32),
            pltpu.SemaphoreType.DMA,
            pltpu.SemaphoreType.DMA,
            pltpu.SemaphoreType.DMA,
            pltpu.SemaphoreType.DMA,
        ],
    )(Au, Bu, srcp, dstp, w2e, w2o, b2v, tempv)


DH = 128
NP = 10240
RPT = NP // NS
EPT = E_PAD // NS
G2 = 64
NCHUNK2 = EPT // G2


DWH = DH // 2


def _s2_body(hs_hbm, src2_hbm, dst_hbm, masks2_hbm, z_hbm,
             agg_hbm,
             spmem, idx_s0, idx_d0, mrow0, urow0, frow0,
             idx_s1, idx_d1, mrow1, urow1, frow1, sem0, sem1):
    c = lax.axis_index("c")
    s = lax.axis_index("s")
    pltpu.sync_copy(z_hbm, spmem.at[pl.ds(s * RPT, RPT)])
    plsc.subcore_barrier()
    ebase = s * EPT
    kcs = [lax.iota(jnp.int32, L) + (16 * j) for j in range(DWH // L)]
    sev = [lax.iota(jnp.int32, L) * 2 + (32 * j) for j in range(DWH // L)]

    def issue(ch, idx_s, idx_d, mrow, urow, sem):
        off = ebase + ch * G2
        pltpu.sync_copy(src2_hbm.at[c, pl.ds(off, G2)], idx_s)
        pltpu.sync_copy(dst_hbm.at[pl.ds(off, G2)], idx_d)
        pltpu.sync_copy(masks2_hbm.at[pl.ds(off // L, G2 // L)], mrow)
        pltpu.async_copy(hs_hbm.at[idx_s], urow, sem)

    def wait(idx_s, urow, sem):
        pltpu.make_async_copy(hs_hbm.at[idx_s], urow, sem).wait()

    def compute(idx_d, mrow, urow, frow):
        @plsc.parallel_loop(0, G2, unroll=2,
                            carry=jnp.zeros((L,), jnp.int32))
        def _(e, ev):
            m = plsc.load_gather(mrow, [ev >> 4, ev & 15])
            for j in range(DWH // L):
                lo, hi = plsc.unpack(
                    plsc.bitcast(plsc.load_gather(urow, [ev, kcs[j]]),
                                 jnp.bfloat16),
                    format=plsc.PackFormat.INTERLEAVED)
                plsc.store_scatter(frow, [ev, sev[j]], lo * m)
                plsc.store_scatter(frow, [ev, sev[j] + 1], hi * m)
            return ev + 1

        pltpu.sync_copy(frow, spmem.at[idx_d], add=True)

    issue(0, idx_s0, idx_d0, mrow0, urow0, sem0)

    def pair(i, carry):
        c0 = 2 * i
        issue(c0 + 1, idx_s1, idx_d1, mrow1, urow1, sem1)
        wait(idx_s0, urow0, sem0)
        compute(idx_d0, mrow0, urow0, frow0)

        @pl.when(c0 + 2 < NCHUNK2)
        def _():
            issue(c0 + 2, idx_s0, idx_d0, mrow0, urow0, sem0)

        wait(idx_s1, urow1, sem1)
        compute(idx_d1, mrow1, urow1, frow1)
        return carry

    lax.fori_loop(0, NCHUNK2 // 2, pair, jnp.int32(0))
    plsc.subcore_barrier()
    pltpu.sync_copy(spmem.at[pl.ds(s * RPT, RPT)],
                    agg_hbm.at[c, pl.ds(s * RPT, RPT)])


def _s2(hflat, src2, dstp, masks2, zrows):
    mesh = plsc.VectorSubcoreMesh(core_axis_name="c", subcore_axis_name="s",
                                  num_cores=NC, num_subcores=NS)
    return pl.kernel(
        _s2_body,
        out_type=jax.ShapeDtypeStruct((NC, NP, DH), jnp.float32),
        mesh=mesh,
        compiler_params=pltpu.CompilerParams(needs_layout_passes=False,
                                             use_tc_tiling_on_sc=False),
        scratch_types=[
            pltpu.VMEM_SHARED((NP, DH), jnp.float32),
            pltpu.VMEM((G2,), jnp.int32),
            pltpu.VMEM((G2,), jnp.int32),
            pltpu.VMEM((G2 // L, L), jnp.float32),
            pltpu.VMEM((G2, DWH), jnp.int32),
            pltpu.VMEM((G2, DH), jnp.float32),
            pltpu.VMEM((G2,), jnp.int32),
            pltpu.VMEM((G2,), jnp.int32),
            pltpu.VMEM((G2 // L, L), jnp.float32),
            pltpu.VMEM((G2, DWH), jnp.int32),
            pltpu.VMEM((G2, DH), jnp.float32),
            pltpu.SemaphoreType.DMA,
            pltpu.SemaphoreType.DMA,
        ],
    )(hflat, src2, dstp, masks2, zrows)


def _t1_body(x_ref, wc_ref, bc_ref, w1a_ref, b1_ref, w1b_ref,
             hs_ref, a_ref, bm_ref):
    h = jax.nn.relu(
        jnp.dot(x_ref[...], wc_ref[...], preferred_element_type=jnp.float32)
        + bc_ref[...][None, :])
    hb = h.astype(jnp.bfloat16)
    hs_ref[0] = hb[:, :DH]
    hs_ref[1] = hb[:, DH:]
    a_ref[...] = (jnp.dot(h, w1a_ref[...], preferred_element_type=jnp.float32)
                  + b1_ref[...][None, :]).astype(jnp.bfloat16)
    bm_ref[...] = jnp.dot(
        h, w1b_ref[...],
        preferred_element_type=jnp.float32).astype(jnp.bfloat16)


def _t1(x, wc, bc, w1a, b1, w1b):
    grid = (N // ROW_BLK,)
    blk = pl.BlockSpec((ROW_BLK, D), lambda i: (i, 0))
    full = pl.BlockSpec((D, D), lambda i: (0, 0))
    vec = pl.BlockSpec((D,), lambda i: (0,))
    return pl.pallas_call(
        _t1_body,
        grid=grid,
        in_specs=[blk, full, vec, full, vec, full],
        out_specs=[pl.BlockSpec((NC, ROW_BLK, DH), lambda i: (0, i, 0)),
                   blk, blk],
        out_shape=[jax.ShapeDtypeStruct((NC, N, DH), jnp.bfloat16),
                   jax.ShapeDtypeStruct((N, D), jnp.bfloat16),
                   jax.ShapeDtypeStruct((N, D), jnp.bfloat16)],
    )(x, wc, bc, w1a, b1, w1b)


def _mid_matmul(a0_ref, a1_ref, w_ref, b_ref):
    return jax.nn.relu(
        jnp.dot(a0_ref[0], w_ref[...][:DH, :],
                preferred_element_type=jnp.float32)
        + jnp.dot(a1_ref[0], w_ref[...][DH:, :],
                  preferred_element_type=jnp.float32)
        + b_ref[...][None, :])


def _t2_body(a0_ref, a1_ref, w_ref, b_ref, hs_ref):
    x = _mid_matmul(a0_ref, a1_ref, w_ref, b_ref).astype(jnp.bfloat16)
    hs_ref[0] = x[:, :DH]
    hs_ref[1] = x[:, DH:]


def _t2(agg, w, b):
    grid = (N // ROW_BLK,)
    return pl.pallas_call(
        _t2_body,
        grid=grid,
        in_specs=[pl.BlockSpec((1, ROW_BLK, DH), lambda i: (0, i, 0)),
                  pl.BlockSpec((1, ROW_BLK, DH), lambda i: (1, i, 0)),
                  pl.BlockSpec((D, D), lambda i: (0, 0)),
                  pl.BlockSpec((D,), lambda i: (0,))],
        out_specs=pl.BlockSpec((NC, ROW_BLK, DH), lambda i: (0, i, 0)),
        out_shape=jax.ShapeDtypeStruct((NC, N, DH), jnp.bfloat16),
    )(agg, agg, w, b)


def _t3_body(a0_ref, a1_ref, w_ref, b_ref, cnt_ref, h_ref, sp_ref):
    h_ref[...] = _mid_matmul(a0_ref, a1_ref, w_ref, b_ref)

    @pl.when(pl.program_id(0) == 0)
    def _():
        sp_ref[0, 0] = jnp.sum(cnt_ref[...]) * (1.0 / E)


def _t3(agg, w, b, cnt):
    grid = (N // ROW_BLK,)
    return pl.pallas_call(
        _t3_body,
        grid=grid,
        in_specs=[pl.BlockSpec((1, ROW_BLK, DH), lambda i: (0, i, 0)),
                  pl.BlockSpec((1, ROW_BLK, DH), lambda i: (1, i, 0)),
                  pl.BlockSpec((D, D), lambda i: (0, 0)),
                  pl.BlockSpec((D,), lambda i: (0,)),
                  pl.BlockSpec((NW, L), lambda i: (0, 0))],
        out_specs=[pl.BlockSpec((ROW_BLK, D), lambda i: (i, 0)),
                   pl.BlockSpec(memory_space=pltpu.MemorySpace.SMEM)],
        out_shape=[jax.ShapeDtypeStruct((N, D), jnp.float32),
                   jax.ShapeDtypeStruct((1, 1), jnp.float32)],
    )(agg, agg, w, b, cnt)


def kernel(node_feats, edge_index, W_ctx, b_ctx, W_imp1, b_imp1, W_imp2,
           b_imp2, mask_temp, W_l0, b_l0, W_l1, b_l1):
    src = edge_index[0]
    dst = edge_index[1]
    w1a = W_imp1[:D, :]
    w1b = W_imp1[D:, :]
    hs, A, B = _t1(node_feats, W_ctx, b_ctx, w1a, b_imp1, w1b)

    pad = jnp.zeros((E_PAD - E,), jnp.int32)
    srcp = jnp.concatenate([src, pad])
    dstp = jnp.concatenate([dst, pad])
    b2v = jnp.broadcast_to(b_imp2, (L,))
    tempv = jnp.broadcast_to(mask_temp, (L,))
    Au = lax.bitcast_convert_type(A.reshape(N, DW, 2), jnp.int32)
    Bu = lax.bitcast_convert_type(B.reshape(N, DW, 2), jnp.int32)
    w2 = W_imp2[:, 0]
    w2e = w2[0::2].reshape(DW // L, L)
    w2o = w2[1::2].reshape(DW // L, L)
    masksP, cnt = _s1(Au, Bu, srcp, dstp, w2e, w2o, b2v, tempv)
    masks = masksP[:E]

    src2 = jnp.stack([srcp, srcp + N])
    masks2 = masksP.reshape(-1, L)
    zrows = jnp.zeros((RPT, DH), jnp.float32)

    def _pack(hsx):
        return lax.bitcast_convert_type(
            hsx.reshape(NC * N, DWH, 2), jnp.int32)

    agg = _s2(_pack(hs), src2, dstp, masks2, zrows)
    hs1 = _t2(agg, W_l0, b_l0)
    agg2 = _s2(_pack(hs1), src2, dstp, masks2, zrows)
    h2, sp = _t3(agg2, W_l1, b_l1, cnt)
    return h2, masks, sp[0, 0]


# speedup vs baseline: 1.0729x; 1.0729x over previous
import functools

import jax
import jax.numpy as jnp
from jax import lax
from jax.experimental import pallas as pl
from jax.experimental.pallas import tpu as pltpu
from jax.experimental.pallas import tpu_sc as plsc

N = 10000
E = 160000
D = 256
ROW_BLK = 400

NC = 2
NS = 16
L = 16
NW = NC * NS
E_PAD = 163840
EPW = E_PAD // NW
G = 128
NCHUNK = EPW // G


DW = D // 2


def _s1_body(a_hbm, b_hbm, src_hbm, dst_hbm, w2e_hbm, w2o_hbm, b2_hbm,
             temp_hbm,
             masks_hbm, cnt_hbm,
             idx_s0, idx_d0, arow0, brow0, idx_s1, idx_d1, arow1, brow1,
             w2e_v, w2o_v, b2_v, temp_v, mask_v, cnt_v,
             sem_a0, sem_b0, sem_a1, sem_b1):
    wid = lax.axis_index("c") * NS + lax.axis_index("s")
    base = wid * EPW
    pltpu.sync_copy(w2e_hbm, w2e_v)
    pltpu.sync_copy(w2o_hbm, w2o_v)
    pltpu.sync_copy(b2_hbm, b2_v)
    pltpu.sync_copy(temp_hbm, temp_v)
    b2 = b2_v[...]
    et = jnp.exp(temp_v[...])
    ngrp = G // L
    eids = [lax.iota(jnp.int32, L) + (g * L) for g in range(ngrp)]

    def issue(c, idx_s, idx_d, arow, brow, sem_a, sem_b):
        off = base + c * G
        pltpu.sync_copy(src_hbm.at[pl.ds(off, G)], idx_s)
        pltpu.sync_copy(dst_hbm.at[pl.ds(off, G)], idx_d)
        pltpu.async_copy(a_hbm.at[idx_s], arow, sem_a)
        pltpu.async_copy(b_hbm.at[idx_d], brow, sem_b)

    def wait(idx_s, idx_d, arow, brow, sem_a, sem_b):
        pltpu.make_async_copy(a_hbm.at[idx_s], arow, sem_a).wait()
        pltpu.make_async_copy(b_hbm.at[idx_d], brow, sem_b).wait()

    def compute(c, arow, brow, cnt):
        off = base + c * G

        init = (tuple(jnp.zeros((L,), jnp.float32) for _ in range(ngrp)),
                jnp.zeros((L,), jnp.int32))

        @plsc.parallel_loop(0, DW, unroll=2, carry=init)
        def accs(k, carry):
            acc, kv = carry
            w2e = plsc.load_gather(w2e_v, [kv >> 4, kv & 15])
            w2o = plsc.load_gather(w2o_v, [kv >> 4, kv & 15])
            nacc = []
            for g in range(ngrp):
                a0, a1 = plsc.unpack(
                    plsc.bitcast(plsc.load_gather(arow, [eids[g], kv]),
                                 jnp.bfloat16),
                    format=plsc.PackFormat.INTERLEAVED)
                b0, b1 = plsc.unpack(
                    plsc.bitcast(plsc.load_gather(brow, [eids[g], kv]),
                                 jnp.bfloat16),
                    format=plsc.PackFormat.INTERLEAVED)
                nacc.append(acc[g]
                            + jnp.maximum(a0 + b0, 0.0) * w2e
                            + jnp.maximum(a1 + b1, 0.0) * w2o)
            return (tuple(nacc), kv + 1)

        accs = accs[0]

        for g in range(ngrp):
            logit = accs[g] + b2
            imp = 1.0 / (1.0 + jnp.exp(-logit))
            m = 1.0 / (1.0 + jnp.exp(-(imp - 0.5) * et))
            gid = off + g * L + lax.iota(jnp.int32, L)
            valid = gid < E
            m = jnp.where(valid, m, 0.0)
            mask_v[pl.ds(g * L, L)] = m
            cnt = cnt + jnp.where(valid & (m < 0.5), 1.0, 0.0)
        pltpu.sync_copy(mask_v, masks_hbm.at[pl.ds(off, G)])
        return cnt

    issue(0, idx_s0, idx_d0, arow0, brow0, sem_a0, sem_b0)

    def pair(i, cnt):
        c0 = 2 * i
        issue(c0 + 1, idx_s1, idx_d1, arow1, brow1, sem_a1, sem_b1)
        wait(idx_s0, idx_d0, arow0, brow0, sem_a0, sem_b0)
        cnt = compute(c0, arow0, brow0, cnt)

        @pl.when(c0 + 2 < NCHUNK)
        def _():
            issue(c0 + 2, idx_s0, idx_d0, arow0, brow0, sem_a0, sem_b0)

        wait(idx_s1, idx_d1, arow1, brow1, sem_a1, sem_b1)
        return compute(c0 + 1, arow1, brow1, cnt)

    cnt = lax.fori_loop(0, NCHUNK // 2, pair,
                        jnp.zeros((L,), jnp.float32))
    cnt_v[...] = cnt
    pltpu.sync_copy(cnt_v, cnt_hbm.at[wid])


def _s1(Au, Bu, srcp, dstp, w2e, w2o, b2v, tempv):
    mesh = plsc.VectorSubcoreMesh(core_axis_name="c", subcore_axis_name="s",
                                  num_cores=NC, num_subcores=NS)
    return pl.kernel(
        _s1_body,
        out_type=[jax.ShapeDtypeStruct((E_PAD,), jnp.float32),
                  jax.ShapeDtypeStruct((NW, L), jnp.float32)],
        mesh=mesh,
        compiler_params=pltpu.CompilerParams(needs_layout_passes=False,
                                             use_tc_tiling_on_sc=False),
        scratch_types=[
            pltpu.VMEM((G,), jnp.int32),
            pltpu.VMEM((G,), jnp.int32),
            pltpu.VMEM((G, DW), jnp.int32),
            pltpu.VMEM((G, DW), jnp.int32),
            pltpu.VMEM((G,), jnp.int32),
            pltpu.VMEM((G,), jnp.int32),
            pltpu.VMEM((G, DW), jnp.int32),
            pltpu.VMEM((G, DW), jnp.int32),
            pltpu.VMEM((DW // L, L), jnp.float32),
            pltpu.VMEM((DW // L, L), jnp.float32),
            pltpu.VMEM((L,), jnp.float32),
            pltpu.VMEM((L,), jnp.float32),
            pltpu.VMEM((G,), jnp.float32),
            pltpu.VMEM((L,), jnp.float32),
            pltpu.SemaphoreType.DMA,
            pltpu.SemaphoreType.DMA,
            pltpu.SemaphoreType.DMA,
            pltpu.SemaphoreType.DMA,
        ],
    )(Au, Bu, srcp, dstp, w2e, w2o, b2v, tempv)


DH = 128
NP = 10240
RPT = NP // NS
EPT = E_PAD // NS
G2 = 128
NCHUNK2 = EPT // G2


DWH = DH // 2


def _s2_body(hs_hbm, src2_hbm, dst_hbm, masks2_hbm, z_hbm,
             agg_hbm,
             spmem, idx_s0, idx_d0, mrow0, urow0,
             idx_s1, idx_d1, mrow1, urow1, frow, sem0, sem1):
    c = lax.axis_index("c")
    s = lax.axis_index("s")
    pltpu.sync_copy(z_hbm, spmem.at[pl.ds(s * RPT, RPT)])
    plsc.subcore_barrier()
    ebase = s * EPT
    kcs = [lax.iota(jnp.int32, L) + (16 * j) for j in range(DWH // L)]
    sev = [lax.iota(jnp.int32, L) * 2 + (32 * j) for j in range(DWH // L)]

    def issue(ch, idx_s, idx_d, mrow, urow, sem):
        off = ebase + ch * G2
        pltpu.sync_copy(src2_hbm.at[c, pl.ds(off, G2)], idx_s)
        pltpu.sync_copy(dst_hbm.at[pl.ds(off, G2)], idx_d)
        pltpu.sync_copy(masks2_hbm.at[pl.ds(off // L, G2 // L)], mrow)
        pltpu.async_copy(hs_hbm.at[idx_s], urow, sem)

    def wait(idx_s, urow, sem):
        pltpu.make_async_copy(hs_hbm.at[idx_s], urow, sem).wait()

    def compute(idx_d, mrow, urow):
        @plsc.parallel_loop(0, G2, unroll=2,
                            carry=jnp.zeros((L,), jnp.int32))
        def _(e, ev):
            m = plsc.load_gather(mrow, [ev >> 4, ev & 15])
            for j in range(DWH // L):
                lo, hi = plsc.unpack(
                    plsc.bitcast(plsc.load_gather(urow, [ev, kcs[j]]),
                                 jnp.bfloat16),
                    format=plsc.PackFormat.INTERLEAVED)
                plsc.store_scatter(frow, [ev, sev[j]], lo * m)
                plsc.store_scatter(frow, [ev, sev[j] + 1], hi * m)
            return ev + 1

        pltpu.sync_copy(frow, spmem.at[idx_d], add=True)

    issue(0, idx_s0, idx_d0, mrow0, urow0, sem0)

    def pair(i, carry):
        c0 = 2 * i
        issue(c0 + 1, idx_s1, idx_d1, mrow1, urow1, sem1)
        wait(idx_s0, urow0, sem0)
        compute(idx_d0, mrow0, urow0)

        @pl.when(c0 + 2 < NCHUNK2)
        def _():
            issue(c0 + 2, idx_s0, idx_d0, mrow0, urow0, sem0)

        wait(idx_s1, urow1, sem1)
        compute(idx_d1, mrow1, urow1)
        return carry

    lax.fori_loop(0, NCHUNK2 // 2, pair, jnp.int32(0))
    plsc.subcore_barrier()
    pltpu.sync_copy(spmem.at[pl.ds(s * RPT, RPT)],
                    agg_hbm.at[c, pl.ds(s * RPT, RPT)])


def _s2(hflat, src2, dstp, masks2, zrows):
    mesh = plsc.VectorSubcoreMesh(core_axis_name="c", subcore_axis_name="s",
                                  num_cores=NC, num_subcores=NS)
    return pl.kernel(
        _s2_body,
        out_type=jax.ShapeDtypeStruct((NC, NP, DH), jnp.float32),
        mesh=mesh,
        compiler_params=pltpu.CompilerParams(needs_layout_passes=False,
                                             use_tc_tiling_on_sc=False),
        scratch_types=[
            pltpu.VMEM_SHARED((NP, DH), jnp.float32),
            pltpu.VMEM((G2,), jnp.int32),
            pltpu.VMEM((G2,), jnp.int32),
            pltpu.VMEM((G2 // L, L), jnp.float32),
            pltpu.VMEM((G2, DWH), jnp.int32),
            pltpu.VMEM((G2,), jnp.int32),
            pltpu.VMEM((G2,), jnp.int32),
            pltpu.VMEM((G2 // L, L), jnp.float32),
            pltpu.VMEM((G2, DWH), jnp.int32),
            pltpu.VMEM((G2, DH), jnp.float32),
            pltpu.SemaphoreType.DMA,
            pltpu.SemaphoreType.DMA,
        ],
    )(hflat, src2, dstp, masks2, zrows)


def _t1_body(x_ref, wc_ref, bc_ref, w1a_ref, b1_ref, w1b_ref,
             hs_ref, a_ref, bm_ref):
    h = jax.nn.relu(
        jnp.dot(x_ref[...], wc_ref[...], preferred_element_type=jnp.float32)
        + bc_ref[...][None, :])
    hb = h.astype(jnp.bfloat16)
    hs_ref[0] = hb[:, :DH]
    hs_ref[1] = hb[:, DH:]
    a_ref[...] = (jnp.dot(h, w1a_ref[...], preferred_element_type=jnp.float32)
                  + b1_ref[...][None, :]).astype(jnp.bfloat16)
    bm_ref[...] = jnp.dot(
        h, w1b_ref[...],
        preferred_element_type=jnp.float32).astype(jnp.bfloat16)


def _t1(x, wc, bc, w1a, b1, w1b):
    grid = (N // ROW_BLK,)
    blk = pl.BlockSpec((ROW_BLK, D), lambda i: (i, 0))
    full = pl.BlockSpec((D, D), lambda i: (0, 0))
    vec = pl.BlockSpec((D,), lambda i: (0,))
    return pl.pallas_call(
        _t1_body,
        grid=grid,
        in_specs=[blk, full, vec, full, vec, full],
        out_specs=[pl.BlockSpec((NC, ROW_BLK, DH), lambda i: (0, i, 0)),
                   blk, blk],
        out_shape=[jax.ShapeDtypeStruct((NC, N, DH), jnp.bfloat16),
                   jax.ShapeDtypeStruct((N, D), jnp.bfloat16),
                   jax.ShapeDtypeStruct((N, D), jnp.bfloat16)],
    )(x, wc, bc, w1a, b1, w1b)


def _mid_matmul(a0_ref, a1_ref, w_ref, b_ref):
    return jax.nn.relu(
        jnp.dot(a0_ref[0], w_ref[...][:DH, :],
                preferred_element_type=jnp.float32)
        + jnp.dot(a1_ref[0], w_ref[...][DH:, :],
                  preferred_element_type=jnp.float32)
        + b_ref[...][None, :])


def _t2_body(a0_ref, a1_ref, w_ref, b_ref, hs_ref):
    x = _mid_matmul(a0_ref, a1_ref, w_ref, b_ref).astype(jnp.bfloat16)
    hs_ref[0] = x[:, :DH]
    hs_ref[1] = x[:, DH:]


def _t2(agg, w, b):
    grid = (N // ROW_BLK,)
    return pl.pallas_call(
        _t2_body,
        grid=grid,
        in_specs=[pl.BlockSpec((1, ROW_BLK, DH), lambda i: (0, i, 0)),
                  pl.BlockSpec((1, ROW_BLK, DH), lambda i: (1, i, 0)),
                  pl.BlockSpec((D, D), lambda i: (0, 0)),
                  pl.BlockSpec((D,), lambda i: (0,))],
        out_specs=pl.BlockSpec((NC, ROW_BLK, DH), lambda i: (0, i, 0)),
        out_shape=jax.ShapeDtypeStruct((NC, N, DH), jnp.bfloat16),
    )(agg, agg, w, b)


def _t3_body(a0_ref, a1_ref, w_ref, b_ref, cnt_ref, h_ref, sp_ref):
    h_ref[...] = _mid_matmul(a0_ref, a1_ref, w_ref, b_ref)

    @pl.when(pl.program_id(0) == 0)
    def _():
        sp_ref[0, 0] = jnp.sum(cnt_ref[...]) * (1.0 / E)


def _t3(agg, w, b, cnt):
    grid = (N // ROW_BLK,)
    return pl.pallas_call(
        _t3_body,
        grid=grid,
        in_specs=[pl.BlockSpec((1, ROW_BLK, DH), lambda i: (0, i, 0)),
                  pl.BlockSpec((1, ROW_BLK, DH), lambda i: (1, i, 0)),
                  pl.BlockSpec((D, D), lambda i: (0, 0)),
                  pl.BlockSpec((D,), lambda i: (0,)),
                  pl.BlockSpec((NW, L), lambda i: (0, 0))],
        out_specs=[pl.BlockSpec((ROW_BLK, D), lambda i: (i, 0)),
                   pl.BlockSpec(memory_space=pltpu.MemorySpace.SMEM)],
        out_shape=[jax.ShapeDtypeStruct((N, D), jnp.float32),
                   jax.ShapeDtypeStruct((1, 1), jnp.float32)],
    )(agg, agg, w, b, cnt)


def kernel(node_feats, edge_index, W_ctx, b_ctx, W_imp1, b_imp1, W_imp2,
           b_imp2, mask_temp, W_l0, b_l0, W_l1, b_l1):
    src = edge_index[0]
    dst = edge_index[1]
    w1a = W_imp1[:D, :]
    w1b = W_imp1[D:, :]
    hs, A, B = _t1(node_feats, W_ctx, b_ctx, w1a, b_imp1, w1b)

    pad = jnp.zeros((E_PAD - E,), jnp.int32)
    srcp = jnp.concatenate([src, pad])
    dstp = jnp.concatenate([dst, pad])
    b2v = jnp.broadcast_to(b_imp2, (L,))
    tempv = jnp.broadcast_to(mask_temp, (L,))
    Au = lax.bitcast_convert_type(A.reshape(N, DW, 2), jnp.int32)
    Bu = lax.bitcast_convert_type(B.reshape(N, DW, 2), jnp.int32)
    w2 = W_imp2[:, 0]
    w2e = w2[0::2].reshape(DW // L, L)
    w2o = w2[1::2].reshape(DW // L, L)
    masksP, cnt = _s1(Au, Bu, srcp, dstp, w2e, w2o, b2v, tempv)
    masks = masksP[:E]

    src2 = jnp.stack([srcp, srcp + N])
    masks2 = masksP.reshape(-1, L)
    zrows = jnp.zeros((RPT, DH), jnp.float32)

    def _pack(hsx):
        return lax.bitcast_convert_type(
            hsx.reshape(NC * N, DWH, 2), jnp.int32)

    agg = _s2(_pack(hs), src2, dstp, masks2, zrows)
    hs1 = _t2(agg, W_l0, b_l0)
    agg2 = _s2(_pack(hs1), src2, dstp, masks2, zrows)
    h2, sp = _t3(agg2, W_l1, b_l1, cnt)
    return h2, masks, sp[0, 0]


# speedup vs baseline: 1.0758x; 1.0027x over previous
import functools

import jax
import jax.numpy as jnp
from jax import lax
from jax.experimental import pallas as pl
from jax.experimental.pallas import tpu as pltpu
from jax.experimental.pallas import tpu_sc as plsc

N = 10000
E = 160000
D = 256
ROW_BLK = 400

NC = 2
NS = 16
L = 16
NW = NC * NS
E_PAD = 163840
EPW = E_PAD // NW
G = 128
NCHUNK = EPW // G


DW = D // 2


def _s1_body(a_hbm, b_hbm, sd_hbm, w2e_hbm, w2o_hbm, b2_hbm,
             temp_hbm,
             masks_hbm, cnt_hbm,
             idx0, arow0, brow0, idx1, arow1, brow1,
             w2e_v, w2o_v, b2_v, temp_v, mask_v, cnt_v,
             sem_a0, sem_b0, sem_a1, sem_b1):
    wid = lax.axis_index("c") * NS + lax.axis_index("s")
    base = wid * EPW
    pltpu.sync_copy(w2e_hbm, w2e_v)
    pltpu.sync_copy(w2o_hbm, w2o_v)
    pltpu.sync_copy(b2_hbm, b2_v)
    pltpu.sync_copy(temp_hbm, temp_v)
    b2 = b2_v[...]
    et = jnp.exp(temp_v[...])
    ngrp = G // L
    eids = [lax.iota(jnp.int32, L) + (g * L) for g in range(ngrp)]

    def issue(c, idx, arow, brow, sem_a, sem_b):
        off = 2 * (base + c * G)
        pltpu.sync_copy(sd_hbm.at[pl.ds(off, 2 * G)], idx)
        pltpu.async_copy(a_hbm.at[idx.at[pl.ds(0, G)]], arow, sem_a)
        pltpu.async_copy(b_hbm.at[idx.at[pl.ds(G, G)]], brow, sem_b)

    def wait(idx, arow, brow, sem_a, sem_b):
        pltpu.make_async_copy(a_hbm.at[idx.at[pl.ds(0, G)]], arow,
                              sem_a).wait()
        pltpu.make_async_copy(b_hbm.at[idx.at[pl.ds(G, G)]], brow,
                              sem_b).wait()

    def compute(c, arow, brow, cnt):
        off = base + c * G

        init = (tuple(jnp.zeros((L,), jnp.float32) for _ in range(ngrp)),
                jnp.zeros((L,), jnp.int32))

        @plsc.parallel_loop(0, DW, unroll=2, carry=init)
        def accs(k, carry):
            acc, kv = carry
            w2e = plsc.load_gather(w2e_v, [kv >> 4, kv & 15])
            w2o = plsc.load_gather(w2o_v, [kv >> 4, kv & 15])
            nacc = []
            for g in range(ngrp):
                a0, a1 = plsc.unpack(
                    plsc.bitcast(plsc.load_gather(arow, [eids[g], kv]),
                                 jnp.bfloat16),
                    format=plsc.PackFormat.INTERLEAVED)
                b0, b1 = plsc.unpack(
                    plsc.bitcast(plsc.load_gather(brow, [eids[g], kv]),
                                 jnp.bfloat16),
                    format=plsc.PackFormat.INTERLEAVED)
                nacc.append(acc[g]
                            + jnp.maximum(a0 + b0, 0.0) * w2e
                            + jnp.maximum(a1 + b1, 0.0) * w2o)
            return (tuple(nacc), kv + 1)

        accs = accs[0]

        for g in range(ngrp):
            logit = accs[g] + b2
            imp = 1.0 / (1.0 + jnp.exp(-logit))
            m = 1.0 / (1.0 + jnp.exp(-(imp - 0.5) * et))
            gid = off + g * L + lax.iota(jnp.int32, L)
            valid = gid < E
            m = jnp.where(valid, m, 0.0)
            mask_v[pl.ds(g * L, L)] = m
            cnt = cnt + jnp.where(valid & (m < 0.5), 1.0, 0.0)
        pltpu.sync_copy(mask_v, masks_hbm.at[pl.ds(off, G)])
        return cnt

    issue(0, idx0, arow0, brow0, sem_a0, sem_b0)

    def pair(i, cnt):
        c0 = 2 * i
        issue(c0 + 1, idx1, arow1, brow1, sem_a1, sem_b1)
        wait(idx0, arow0, brow0, sem_a0, sem_b0)
        cnt = compute(c0, arow0, brow0, cnt)

        @pl.when(c0 + 2 < NCHUNK)
        def _():
            issue(c0 + 2, idx0, arow0, brow0, sem_a0, sem_b0)

        wait(idx1, arow1, brow1, sem_a1, sem_b1)
        return compute(c0 + 1, arow1, brow1, cnt)

    cnt = lax.fori_loop(0, NCHUNK // 2, pair,
                        jnp.zeros((L,), jnp.float32))
    cnt_v[...] = cnt
    pltpu.sync_copy(cnt_v, cnt_hbm.at[wid])


def _s1(Au, Bu, sd, w2e, w2o, b2v, tempv):
    mesh = plsc.VectorSubcoreMesh(core_axis_name="c", subcore_axis_name="s",
                                  num_cores=NC, num_subcores=NS)
    return pl.kernel(
        _s1_body,
        out_type=[jax.ShapeDtypeStruct((E_PAD,), jnp.float32),
                  jax.ShapeDtypeStruct((NW, L), jnp.float32)],
        mesh=mesh,
        compiler_params=pltpu.CompilerParams(needs_layout_passes=False,
                                             use_tc_tiling_on_sc=False),
        scratch_types=[
            pltpu.VMEM((2 * G,), jnp.int32),
            pltpu.VMEM((G, DW), jnp.int32),
            pltpu.VMEM((G, DW), jnp.int32),
            pltpu.VMEM((2 * G,), jnp.int32),
            pltpu.VMEM((G, DW), jnp.int32),
            pltpu.VMEM((G, DW), jnp.int32),
            pltpu.VMEM((DW // L, L), jnp.float32),
            pltpu.VMEM((DW // L, L), jnp.float32),
            pltpu.VMEM((L,), jnp.float32),
            pltpu.VMEM((L,), jnp.float32),
            pltpu.VMEM((G,), jnp.float32),
            pltpu.VMEM((L,), jnp.float32),
            pltpu.SemaphoreType.DMA,
            pltpu.SemaphoreType.DMA,
            pltpu.SemaphoreType.DMA,
            pltpu.SemaphoreType.DMA,
        ],
    )(Au, Bu, sd, w2e, w2o, b2v, tempv)


DH = 128
NP = 10240
RPT = NP // NS
EPT = E_PAD // NS
G2 = 128
NCHUNK2 = EPT // G2


DWH = DH // 2


def _s2_body(hs_hbm, src2_hbm, dst_hbm, masks2_hbm, z_hbm,
             agg_hbm,
             spmem, idx_s0, idx_d0, mrow0, urow0,
             idx_s1, idx_d1, mrow1, urow1, frow, sem0, sem1):
    c = lax.axis_index("c")
    s = lax.axis_index("s")
    pltpu.sync_copy(z_hbm, spmem.at[pl.ds(s * RPT, RPT)])
    plsc.subcore_barrier()
    ebase = s * EPT
    kcs = [lax.iota(jnp.int32, L) + (16 * j) for j in range(DWH // L)]
    sev = [lax.iota(jnp.int32, L) * 2 + (32 * j) for j in range(DWH // L)]

    def issue(ch, idx_s, idx_d, mrow, urow, sem):
        off = ebase + ch * G2
        pltpu.sync_copy(src2_hbm.at[c, pl.ds(off, G2)], idx_s)
        pltpu.sync_copy(dst_hbm.at[pl.ds(off, G2)], idx_d)
        pltpu.sync_copy(masks2_hbm.at[pl.ds(off // L, G2 // L)], mrow)
        pltpu.async_copy(hs_hbm.at[idx_s], urow, sem)

    def wait(idx_s, urow, sem):
        pltpu.make_async_copy(hs_hbm.at[idx_s], urow, sem).wait()

    def compute(idx_d, mrow, urow):
        @plsc.parallel_loop(0, G2, unroll=2,
                            carry=jnp.zeros((L,), jnp.int32))
        def _(e, ev):
            m = plsc.load_gather(mrow, [ev >> 4, ev & 15])
            for j in range(DWH // L):
                lo, hi = plsc.unpack(
                    plsc.bitcast(plsc.load_gather(urow, [ev, kcs[j]]),
                                 jnp.bfloat16),
                    format=plsc.PackFormat.INTERLEAVED)
                plsc.store_scatter(frow, [ev, sev[j]], lo * m)
                plsc.store_scatter(frow, [ev, sev[j] + 1], hi * m)
            return ev + 1

        pltpu.sync_copy(frow, spmem.at[idx_d], add=True)

    issue(0, idx_s0, idx_d0, mrow0, urow0, sem0)

    def pair(i, carry):
        c0 = 2 * i
        issue(c0 + 1, idx_s1, idx_d1, mrow1, urow1, sem1)
        wait(idx_s0, urow0, sem0)
        compute(idx_d0, mrow0, urow0)

        @pl.when(c0 + 2 < NCHUNK2)
        def _():
            issue(c0 + 2, idx_s0, idx_d0, mrow0, urow0, sem0)

        wait(idx_s1, urow1, sem1)
        compute(idx_d1, mrow1, urow1)
        return carry

    lax.fori_loop(0, NCHUNK2 // 2, pair, jnp.int32(0))
    plsc.subcore_barrier()
    pltpu.sync_copy(spmem.at[pl.ds(s * RPT, RPT)],
                    agg_hbm.at[c, pl.ds(s * RPT, RPT)])


def _s2(hflat, src2, dstp, masks2, zrows):
    mesh = plsc.VectorSubcoreMesh(core_axis_name="c", subcore_axis_name="s",
                                  num_cores=NC, num_subcores=NS)
    return pl.kernel(
        _s2_body,
        out_type=jax.ShapeDtypeStruct((NC, NP, DH), jnp.float32),
        mesh=mesh,
        compiler_params=pltpu.CompilerParams(needs_layout_passes=False,
                                             use_tc_tiling_on_sc=False),
        scratch_types=[
            pltpu.VMEM_SHARED((NP, DH), jnp.float32),
            pltpu.VMEM((G2,), jnp.int32),
            pltpu.VMEM((G2,), jnp.int32),
            pltpu.VMEM((G2 // L, L), jnp.float32),
            pltpu.VMEM((G2, DWH), jnp.int32),
            pltpu.VMEM((G2,), jnp.int32),
            pltpu.VMEM((G2,), jnp.int32),
            pltpu.VMEM((G2 // L, L), jnp.float32),
            pltpu.VMEM((G2, DWH), jnp.int32),
            pltpu.VMEM((G2, DH), jnp.float32),
            pltpu.SemaphoreType.DMA,
            pltpu.SemaphoreType.DMA,
        ],
    )(hflat, src2, dstp, masks2, zrows)


def _t1_body(x_ref, wc_ref, bc_ref, w1a_ref, b1_ref, w1b_ref,
             hs_ref, a_ref, bm_ref):
    h = jax.nn.relu(
        jnp.dot(x_ref[...], wc_ref[...], preferred_element_type=jnp.float32)
        + bc_ref[...][None, :])
    hb = h.astype(jnp.bfloat16)
    hs_ref[0] = hb[:, :DH]
    hs_ref[1] = hb[:, DH:]
    a_ref[...] = (jnp.dot(h, w1a_ref[...], preferred_element_type=jnp.float32)
                  + b1_ref[...][None, :]).astype(jnp.bfloat16)
    bm_ref[...] = jnp.dot(
        h, w1b_ref[...],
        preferred_element_type=jnp.float32).astype(jnp.bfloat16)


def _t1(x, wc, bc, w1a, b1, w1b):
    grid = (N // ROW_BLK,)
    blk = pl.BlockSpec((ROW_BLK, D), lambda i: (i, 0))
    full = pl.BlockSpec((D, D), lambda i: (0, 0))
    vec = pl.BlockSpec((D,), lambda i: (0,))
    return pl.pallas_call(
        _t1_body,
        grid=grid,
        in_specs=[blk, full, vec, full, vec, full],
        out_specs=[pl.BlockSpec((NC, ROW_BLK, DH), lambda i: (0, i, 0)),
                   blk, blk],
        out_shape=[jax.ShapeDtypeStruct((NC, N, DH), jnp.bfloat16),
                   jax.ShapeDtypeStruct((N, D), jnp.bfloat16),
                   jax.ShapeDtypeStruct((N, D), jnp.bfloat16)],
    )(x, wc, bc, w1a, b1, w1b)


def _mid_matmul(a0_ref, a1_ref, w_ref, b_ref):
    return jax.nn.relu(
        jnp.dot(a0_ref[0], w_ref[...][:DH, :],
                preferred_element_type=jnp.float32)
        + jnp.dot(a1_ref[0], w_ref[...][DH:, :],
                  preferred_element_type=jnp.float32)
        + b_ref[...][None, :])


def _t2_body(a0_ref, a1_ref, w_ref, b_ref, hs_ref):
    x = _mid_matmul(a0_ref, a1_ref, w_ref, b_ref).astype(jnp.bfloat16)
    hs_ref[0] = x[:, :DH]
    hs_ref[1] = x[:, DH:]


def _t2(agg, w, b):
    grid = (N // ROW_BLK,)
    return pl.pallas_call(
        _t2_body,
        grid=grid,
        in_specs=[pl.BlockSpec((1, ROW_BLK, DH), lambda i: (0, i, 0)),
                  pl.BlockSpec((1, ROW_BLK, DH), lambda i: (1, i, 0)),
                  pl.BlockSpec((D, D), lambda i: (0, 0)),
                  pl.BlockSpec((D,), lambda i: (0,))],
        out_specs=pl.BlockSpec((NC, ROW_BLK, DH), lambda i: (0, i, 0)),
        out_shape=jax.ShapeDtypeStruct((NC, N, DH), jnp.bfloat16),
    )(agg, agg, w, b)


def _t3_body(a0_ref, a1_ref, w_ref, b_ref, cnt_ref, h_ref, sp_ref):
    h_ref[...] = _mid_matmul(a0_ref, a1_ref, w_ref, b_ref)

    @pl.when(pl.program_id(0) == 0)
    def _():
        sp_ref[0, 0] = jnp.sum(cnt_ref[...]) * (1.0 / E)


def _t3(agg, w, b, cnt):
    grid = (N // ROW_BLK,)
    return pl.pallas_call(
        _t3_body,
        grid=grid,
        in_specs=[pl.BlockSpec((1, ROW_BLK, DH), lambda i: (0, i, 0)),
                  pl.BlockSpec((1, ROW_BLK, DH), lambda i: (1, i, 0)),
                  pl.BlockSpec((D, D), lambda i: (0, 0)),
                  pl.BlockSpec((D,), lambda i: (0,)),
                  pl.BlockSpec((NW, L), lambda i: (0, 0))],
        out_specs=[pl.BlockSpec((ROW_BLK, D), lambda i: (i, 0)),
                   pl.BlockSpec(memory_space=pltpu.MemorySpace.SMEM)],
        out_shape=[jax.ShapeDtypeStruct((N, D), jnp.float32),
                   jax.ShapeDtypeStruct((1, 1), jnp.float32)],
    )(agg, agg, w, b, cnt)


def kernel(node_feats, edge_index, W_ctx, b_ctx, W_imp1, b_imp1, W_imp2,
           b_imp2, mask_temp, W_l0, b_l0, W_l1, b_l1):
    src = edge_index[0]
    dst = edge_index[1]
    w1a = W_imp1[:D, :]
    w1b = W_imp1[D:, :]
    hs, A, B = _t1(node_feats, W_ctx, b_ctx, w1a, b_imp1, w1b)

    pad = jnp.zeros((E_PAD - E,), jnp.int32)
    srcp = jnp.concatenate([src, pad])
    dstp = jnp.concatenate([dst, pad])
    b2v = jnp.broadcast_to(b_imp2, (L,))
    tempv = jnp.broadcast_to(mask_temp, (L,))
    Au = lax.bitcast_convert_type(A.reshape(N, DW, 2), jnp.int32)
    Bu = lax.bitcast_convert_type(B.reshape(N, DW, 2), jnp.int32)
    w2 = W_imp2[:, 0]
    w2e = w2[0::2].reshape(DW // L, L)
    w2o = w2[1::2].reshape(DW // L, L)
    sd = jnp.stack([srcp.reshape(-1, G), dstp.reshape(-1, G)],
                   axis=1).reshape(-1)
    masksP, cnt = _s1(Au, Bu, sd, w2e, w2o, b2v, tempv)
    masks = masksP[:E]

    src2 = jnp.stack([srcp, srcp + N])
    masks2 = masksP.reshape(-1, L)
    zrows = jnp.zeros((RPT, DH), jnp.float32)

    def _pack(hsx):
        return lax.bitcast_convert_type(
            hsx.reshape(NC * N, DWH, 2), jnp.int32)

    agg = _s2(_pack(hs), src2, dstp, masks2, zrows)
    hs1 = _t2(agg, W_l0, b_l0)
    agg2 = _s2(_pack(hs1), src2, dstp, masks2, zrows)
    h2, sp = _t3(agg2, W_l1, b_l1, cnt)
    return h2, masks, sp[0, 0]


# speedup vs baseline: 1.0759x; 1.0001x over previous
import jax
import jax.numpy as jnp
from jax import lax
from jax.experimental import pallas as pl
from jax.experimental.pallas import tpu as pltpu
from jax.experimental.pallas import tpu_sc as plsc

N = 10000
E = 160000
D = 256
ROW_BLK = 400

NC = 2
NS = 16
L = 16
NW = NC * NS
E_PAD = 163840
EPW = E_PAD // NW
G = 128
NCHUNK = EPW // G


DW = D // 2


def _s1_body(a_hbm, b_hbm, sd_hbm, w2e_hbm, w2o_hbm, b2_hbm,
             temp_hbm,
             masks_hbm, cnt_hbm,
             idx0, arow0, brow0, idx1, arow1, brow1,
             w2e_v, w2o_v, b2_v, temp_v, mask_v, cnt_v,
             sem_a0, sem_b0, sem_a1, sem_b1):
    wid = lax.axis_index("c") * NS + lax.axis_index("s")
    base = wid * EPW
    pltpu.sync_copy(w2e_hbm, w2e_v)
    pltpu.sync_copy(w2o_hbm, w2o_v)
    pltpu.sync_copy(b2_hbm, b2_v)
    pltpu.sync_copy(temp_hbm, temp_v)
    b2 = b2_v[...]
    et = jnp.exp(temp_v[...])
    ngrp = G // L
    eids = [lax.iota(jnp.int32, L) + (g * L) for g in range(ngrp)]

    def issue(c, idx, arow, brow, sem_a, sem_b):
        off = 2 * (base + c * G)
        pltpu.sync_copy(sd_hbm.at[pl.ds(off, 2 * G)], idx)
        pltpu.async_copy(a_hbm.at[idx.at[pl.ds(0, G)]], arow, sem_a)
        pltpu.async_copy(b_hbm.at[idx.at[pl.ds(G, G)]], brow, sem_b)

    def wait(idx, arow, brow, sem_a, sem_b):
        pltpu.make_async_copy(a_hbm.at[idx.at[pl.ds(0, G)]], arow,
                              sem_a).wait()
        pltpu.make_async_copy(b_hbm.at[idx.at[pl.ds(G, G)]], brow,
                              sem_b).wait()

    def compute(c, arow, brow, cnt):
        off = base + c * G

        init = (tuple(jnp.zeros((L,), jnp.float32) for _ in range(ngrp)),
                jnp.zeros((L,), jnp.int32))

        @plsc.parallel_loop(0, DW, unroll=2, carry=init)
        def accs(k, carry):
            acc, kv = carry
            w2e = plsc.load_gather(w2e_v, [kv >> 4, kv & 15])
            w2o = plsc.load_gather(w2o_v, [kv >> 4, kv & 15])
            nacc = []
            for g in range(ngrp):
                a0, a1 = plsc.unpack(
                    plsc.bitcast(plsc.load_gather(arow, [eids[g], kv]),
                                 jnp.bfloat16),
                    format=plsc.PackFormat.INTERLEAVED)
                b0, b1 = plsc.unpack(
                    plsc.bitcast(plsc.load_gather(brow, [eids[g], kv]),
                                 jnp.bfloat16),
                    format=plsc.PackFormat.INTERLEAVED)
                nacc.append(acc[g]
                            + jnp.maximum(a0 + b0, 0.0) * w2e
                            + jnp.maximum(a1 + b1, 0.0) * w2o)
            return (tuple(nacc), kv + 1)

        accs = accs[0]

        for g in range(ngrp):
            logit = accs[g] + b2
            imp = 1.0 / (1.0 + jnp.exp(-logit))
            m = 1.0 / (1.0 + jnp.exp(-(imp - 0.5) * et))
            gid = off + g * L + lax.iota(jnp.int32, L)
            valid = gid < E
            m = jnp.where(valid, m, 0.0)
            mask_v[pl.ds(g * L, L)] = m
            cnt = cnt + jnp.where(valid & (m < 0.5), 1.0, 0.0)
        pltpu.sync_copy(mask_v, masks_hbm.at[pl.ds(off, G)])
        return cnt

    issue(0, idx0, arow0, brow0, sem_a0, sem_b0)

    def pair(i, cnt):
        c0 = 2 * i
        issue(c0 + 1, idx1, arow1, brow1, sem_a1, sem_b1)
        wait(idx0, arow0, brow0, sem_a0, sem_b0)
        cnt = compute(c0, arow0, brow0, cnt)

        @pl.when(c0 + 2 < NCHUNK)
        def _():
            issue(c0 + 2, idx0, arow0, brow0, sem_a0, sem_b0)

        wait(idx1, arow1, brow1, sem_a1, sem_b1)
        return compute(c0 + 1, arow1, brow1, cnt)

    cnt = lax.fori_loop(0, NCHUNK // 2, pair,
                        jnp.zeros((L,), jnp.float32))
    cnt_v[...] = cnt
    pltpu.sync_copy(cnt_v, cnt_hbm.at[wid])


def _s1(Au, Bu, sd, w2e, w2o, b2v, tempv):
    mesh = plsc.VectorSubcoreMesh(core_axis_name="c", subcore_axis_name="s",
                                  num_cores=NC, num_subcores=NS)
    return pl.kernel(
        _s1_body,
        out_type=[jax.ShapeDtypeStruct((E_PAD,), jnp.float32),
                  jax.ShapeDtypeStruct((NW, L), jnp.float32)],
        mesh=mesh,
        compiler_params=pltpu.CompilerParams(needs_layout_passes=False,
                                             use_tc_tiling_on_sc=False),
        scratch_types=[
            pltpu.VMEM((2 * G,), jnp.int32),
            pltpu.VMEM((G, DW), jnp.int32),
            pltpu.VMEM((G, DW), jnp.int32),
            pltpu.VMEM((2 * G,), jnp.int32),
            pltpu.VMEM((G, DW), jnp.int32),
            pltpu.VMEM((G, DW), jnp.int32),
            pltpu.VMEM((DW // L, L), jnp.float32),
            pltpu.VMEM((DW // L, L), jnp.float32),
            pltpu.VMEM((L,), jnp.float32),
            pltpu.VMEM((L,), jnp.float32),
            pltpu.VMEM((G,), jnp.float32),
            pltpu.VMEM((L,), jnp.float32),
            pltpu.SemaphoreType.DMA,
            pltpu.SemaphoreType.DMA,
            pltpu.SemaphoreType.DMA,
            pltpu.SemaphoreType.DMA,
        ],
    )(Au, Bu, sd, w2e, w2o, b2v, tempv)


DH = 128
NP = 10240
RPT = NP // NS
EPT = E_PAD // NS
G2 = 128
NCHUNK2 = EPT // G2


DWH = DH // 2


def _s2_body(hs_hbm, src2_hbm, dst_hbm, masks2_hbm, z_hbm,
             agg_hbm,
             spmem, idx_s0, idx_d0, mrow0, urow0,
             idx_s1, idx_d1, mrow1, urow1, frow, sem0, sem1):
    c = lax.axis_index("c")
    s = lax.axis_index("s")
    pltpu.sync_copy(z_hbm, spmem.at[pl.ds(s * RPT, RPT)])
    plsc.subcore_barrier()
    ebase = s * EPT
    kcs = [lax.iota(jnp.int32, L) + (16 * j) for j in range(DWH // L)]
    sev = [lax.iota(jnp.int32, L) * 2 + (32 * j) for j in range(DWH // L)]

    def issue(ch, idx_s, idx_d, mrow, urow, sem):
        off = ebase + ch * G2
        pltpu.sync_copy(src2_hbm.at[c, pl.ds(off, G2)], idx_s)
        pltpu.sync_copy(dst_hbm.at[pl.ds(off, G2)], idx_d)
        pltpu.sync_copy(masks2_hbm.at[pl.ds(off // L, G2 // L)], mrow)
        pltpu.async_copy(hs_hbm.at[idx_s], urow, sem)

    def wait(idx_s, urow, sem):
        pltpu.make_async_copy(hs_hbm.at[idx_s], urow, sem).wait()

    def compute(idx_d, mrow, urow):
        @plsc.parallel_loop(0, G2, unroll=2,
                            carry=jnp.zeros((L,), jnp.int32))
        def _(e, ev):
            m = plsc.load_gather(mrow, [ev >> 4, ev & 15])
            for j in range(DWH // L):
                lo, hi = plsc.unpack(
                    plsc.bitcast(plsc.load_gather(urow, [ev, kcs[j]]),
                                 jnp.bfloat16),
                    format=plsc.PackFormat.INTERLEAVED)
                plsc.store_scatter(frow, [ev, sev[j]], lo * m)
                plsc.store_scatter(frow, [ev, sev[j] + 1], hi * m)
            return ev + 1

        pltpu.sync_copy(frow, spmem.at[idx_d], add=True)

    issue(0, idx_s0, idx_d0, mrow0, urow0, sem0)

    def pair(i, carry):
        c0 = 2 * i
        issue(c0 + 1, idx_s1, idx_d1, mrow1, urow1, sem1)
        wait(idx_s0, urow0, sem0)
        compute(idx_d0, mrow0, urow0)

        @pl.when(c0 + 2 < NCHUNK2)
        def _():
            issue(c0 + 2, idx_s0, idx_d0, mrow0, urow0, sem0)

        wait(idx_s1, urow1, sem1)
        compute(idx_d1, mrow1, urow1)
        return carry

    lax.fori_loop(0, NCHUNK2 // 2, pair, jnp.int32(0))
    plsc.subcore_barrier()
    pltpu.sync_copy(spmem.at[pl.ds(s * RPT, RPT)],
                    agg_hbm.at[c, pl.ds(s * RPT, RPT)])


def _s2(hflat, src2, dstp, masks2, zrows):
    mesh = plsc.VectorSubcoreMesh(core_axis_name="c", subcore_axis_name="s",
                                  num_cores=NC, num_subcores=NS)
    return pl.kernel(
        _s2_body,
        out_type=jax.ShapeDtypeStruct((NC, NP, DH), jnp.float32),
        mesh=mesh,
        compiler_params=pltpu.CompilerParams(needs_layout_passes=False,
                                             use_tc_tiling_on_sc=False),
        scratch_types=[
            pltpu.VMEM_SHARED((NP, DH), jnp.float32),
            pltpu.VMEM((G2,), jnp.int32),
            pltpu.VMEM((G2,), jnp.int32),
            pltpu.VMEM((G2 // L, L), jnp.float32),
            pltpu.VMEM((G2, DWH), jnp.int32),
            pltpu.VMEM((G2,), jnp.int32),
            pltpu.VMEM((G2,), jnp.int32),
            pltpu.VMEM((G2 // L, L), jnp.float32),
            pltpu.VMEM((G2, DWH), jnp.int32),
            pltpu.VMEM((G2, DH), jnp.float32),
            pltpu.SemaphoreType.DMA,
            pltpu.SemaphoreType.DMA,
        ],
    )(hflat, src2, dstp, masks2, zrows)


def _t1_body(x_ref, wc_ref, bc_ref, w1a_ref, b1_ref, w1b_ref,
             hs_ref, a_ref, bm_ref):
    h = jax.nn.relu(
        jnp.dot(x_ref[...], wc_ref[...], preferred_element_type=jnp.float32)
        + bc_ref[...][None, :])
    hb = h.astype(jnp.bfloat16)
    hs_ref[0] = hb[:, :DH]
    hs_ref[1] = hb[:, DH:]
    a_ref[...] = (jnp.dot(h, w1a_ref[...], preferred_element_type=jnp.float32)
                  + b1_ref[...][None, :]).astype(jnp.bfloat16)
    bm_ref[...] = jnp.dot(
        h, w1b_ref[...],
        preferred_element_type=jnp.float32).astype(jnp.bfloat16)


def _t1(x, wc, bc, w1a, b1, w1b):
    grid = (N // ROW_BLK,)
    blk = pl.BlockSpec((ROW_BLK, D), lambda i: (i, 0))
    full = pl.BlockSpec((D, D), lambda i: (0, 0))
    vec = pl.BlockSpec((D,), lambda i: (0,))
    return pl.pallas_call(
        _t1_body,
        grid=grid,
        in_specs=[blk, full, vec, full, vec, full],
        out_specs=[pl.BlockSpec((NC, ROW_BLK, DH), lambda i: (0, i, 0)),
                   blk, blk],
        out_shape=[jax.ShapeDtypeStruct((NC, N, DH), jnp.bfloat16),
                   jax.ShapeDtypeStruct((N, D), jnp.bfloat16),
                   jax.ShapeDtypeStruct((N, D), jnp.bfloat16)],
    )(x, wc, bc, w1a, b1, w1b)


def _mid_matmul(a0_ref, a1_ref, w_ref, b_ref):
    return jax.nn.relu(
        jnp.dot(a0_ref[0], w_ref[...][:DH, :],
                preferred_element_type=jnp.float32)
        + jnp.dot(a1_ref[0], w_ref[...][DH:, :],
                  preferred_element_type=jnp.float32)
        + b_ref[...][None, :])


def _t2_body(a0_ref, a1_ref, w_ref, b_ref, hs_ref):
    x = _mid_matmul(a0_ref, a1_ref, w_ref, b_ref).astype(jnp.bfloat16)
    hs_ref[0] = x[:, :DH]
    hs_ref[1] = x[:, DH:]


def _t2(agg, w, b):
    grid = (N // ROW_BLK,)
    return pl.pallas_call(
        _t2_body,
        grid=grid,
        in_specs=[pl.BlockSpec((1, ROW_BLK, DH), lambda i: (0, i, 0)),
                  pl.BlockSpec((1, ROW_BLK, DH), lambda i: (1, i, 0)),
                  pl.BlockSpec((D, D), lambda i: (0, 0)),
                  pl.BlockSpec((D,), lambda i: (0,))],
        out_specs=pl.BlockSpec((NC, ROW_BLK, DH), lambda i: (0, i, 0)),
        out_shape=jax.ShapeDtypeStruct((NC, N, DH), jnp.bfloat16),
    )(agg, agg, w, b)


def _t3_body(a0_ref, a1_ref, w_ref, b_ref, cnt_ref, h_ref, sp_ref):
    h_ref[...] = _mid_matmul(a0_ref, a1_ref, w_ref, b_ref)

    @pl.when(pl.program_id(0) == 0)
    def _():
        sp_ref[0, 0] = jnp.sum(cnt_ref[...]) * (1.0 / E)


def _t3(agg, w, b, cnt):
    grid = (N // ROW_BLK,)
    return pl.pallas_call(
        _t3_body,
        grid=grid,
        in_specs=[pl.BlockSpec((1, ROW_BLK, DH), lambda i: (0, i, 0)),
                  pl.BlockSpec((1, ROW_BLK, DH), lambda i: (1, i, 0)),
                  pl.BlockSpec((D, D), lambda i: (0, 0)),
                  pl.BlockSpec((D,), lambda i: (0,)),
                  pl.BlockSpec((NW, L), lambda i: (0, 0))],
        out_specs=[pl.BlockSpec((ROW_BLK, D), lambda i: (i, 0)),
                   pl.BlockSpec(memory_space=pltpu.MemorySpace.SMEM)],
        out_shape=[jax.ShapeDtypeStruct((N, D), jnp.float32),
                   jax.ShapeDtypeStruct((1, 1), jnp.float32)],
    )(agg, agg, w, b, cnt)


def kernel(node_feats, edge_index, W_ctx, b_ctx, W_imp1, b_imp1, W_imp2,
           b_imp2, mask_temp, W_l0, b_l0, W_l1, b_l1):
    src = edge_index[0]
    dst = edge_index[1]
    w1a = W_imp1[:D, :]
    w1b = W_imp1[D:, :]
    hs, A, B = _t1(node_feats, W_ctx, b_ctx, w1a, b_imp1, w1b)

    pad = jnp.zeros((E_PAD - E,), jnp.int32)
    srcp = jnp.concatenate([src, pad])
    dstp = jnp.concatenate([dst, pad])
    b2v = jnp.broadcast_to(b_imp2, (L,))
    tempv = jnp.broadcast_to(mask_temp, (L,))
    Au = lax.bitcast_convert_type(A.reshape(N, DW, 2), jnp.int32)
    Bu = lax.bitcast_convert_type(B.reshape(N, DW, 2), jnp.int32)
    w2 = W_imp2[:, 0]
    w2e = w2[0::2].reshape(DW // L, L)
    w2o = w2[1::2].reshape(DW // L, L)
    sd = jnp.stack([srcp.reshape(-1, G), dstp.reshape(-1, G)],
                   axis=1).reshape(-1)
    masksP, cnt = _s1(Au, Bu, sd, w2e, w2o, b2v, tempv)
    masks = masksP[:E]

    src2 = jnp.stack([srcp, srcp + N])
    masks2 = masksP.reshape(-1, L)
    zrows = jnp.zeros((RPT, DH), jnp.float32)

    def _pack(hsx):
        return lax.bitcast_convert_type(
            hsx.reshape(NC * N, DWH, 2), jnp.int32)

    agg = _s2(_pack(hs), src2, dstp, masks2, zrows)
    hs1 = _t2(agg, W_l0, b_l0)
    agg2 = _s2(_pack(hs1), src2, dstp, masks2, zrows)
    h2, sp = _t3(agg2, W_l1, b_l1, cnt)
    return h2, masks, sp[0, 0]
